# baseline (speedup 1.0000x reference)
import functools

import jax
import jax.numpy as jnp
from jax import lax
from jax.experimental import pallas as pl
from jax.experimental.pallas import tpu as pltpu
from jax.experimental.pallas import tpu_sc as plsc

N = 10000
E = 320000
D = 128
L = 16
NC = 2
NS = 16
NW = NC * NS
CHUNK = 128
FAST_CORE = 0
NCHT = 158
EPAD = NS * NCHT * CHUNK
DRAIN = 80
NDC = N // DRAIN


def _mm_body(x_ref, w_ref, o_ref):
    o_ref[...] = jnp.dot(x_ref[...], w_ref[...],
                         preferred_element_type=jnp.float32)


def _matmul(x, W):
    return pl.pallas_call(
        _mm_body,
        grid=(10,),
        in_specs=[
            pl.BlockSpec((N // 10, D), lambda i: (i, 0)),
            pl.BlockSpec((D, D), lambda i: (0, 0)),
        ],
        out_specs=pl.BlockSpec((N // 10, D), lambda i: (i, 0)),
        out_shape=jax.ShapeDtypeStruct((N, D), jnp.float32),
    )(x, W)


def _add_body(a_ref, b_ref, o_ref):
    o_ref[...] = a_ref[...] + b_ref[...]


def _add(a, b):
    return pl.pallas_call(
        _add_body,
        grid=(10,),
        in_specs=[
            pl.BlockSpec((N // 10, D), lambda i: (i, 0)),
            pl.BlockSpec((N // 10, D), lambda i: (i, 0)),
        ],
        out_specs=pl.BlockSpec((N // 10, D), lambda i: (i, 0)),
        out_shape=jax.ShapeDtypeStruct((N, D), jnp.float32),
    )(a, b)


def _sc_body(support, row3, col3, val3, out, acc,
             cidx0, cidx1, ridx0, ridx1, vals0, vals1, rows0, rows1,
             obuf, semi0, semi1, semg0, semg1):
    c = lax.axis_index("c")
    s = lax.axis_index("s")
    on = c == FAST_CORE
    nch = jnp.where(on, NCHT, 0)
    base = s * NCHT
    cidx = (cidx0, cidx1)
    ridx = (ridx0, ridx1)
    vals = (vals0, vals1)
    rows = (rows0, rows1)
    semi = (semi0, semi1)
    semg = (semg0, semg1)

    def _issue_idx(k, b):
        off = base + k
        pltpu.async_copy(col3.at[off, 0], cidx[b], semi[b])
        pltpu.async_copy(row3.at[off, 0], ridx[b], semi[b])
        pltpu.async_copy(val3.at[off, 0], vals[b], semi[b])

    def _wait_idx(b):
        pltpu.make_async_copy(col3.at[0, 0], cidx[b], semi[b]).wait()
        pltpu.make_async_copy(row3.at[0, 0], ridx[b], semi[b]).wait()
        pltpu.make_async_copy(val3.at[0, 0], vals[b], semi[b]).wait()

    def _issue_gather(b):
        pltpu.async_copy(support.at[cidx[b]], rows[b], semg[b])

    def _wait_gather(b):
        pltpu.make_async_copy(support.at[cidx[b]], rows[b], semg[b]).wait()

    @pl.when(on)
    def _():
        _issue_idx(0, 0)
        _issue_idx(1, 1)
        _wait_idx(0)
        _issue_gather(0)

    zero = jnp.zeros((L,), jnp.float32)

    def _zrow(r, _):
        for j in range(D // L):
            obuf[r, pl.ds(j * L, L)] = zero
        return 0

    @pl.when(on)
    def _():
        lax.fori_loop(0, DRAIN, _zrow, 0)
        for t in range(-(-NDC // NS)):
            idx = s + t * NS

            @pl.when(idx < NDC)
            def _():
                pltpu.sync_copy(obuf, acc.at[pl.ds(idx * DRAIN, DRAIN)])

    plsc.subcore_barrier()

    def _pair(i, _):
        for u in range(2):
            k = 2 * i + u
            b = u
            nb = 1 - u
            _wait_gather(b)

            @pl.when(k + 1 < nch)
            def _():
                _wait_idx(nb)
                _issue_gather(nb)

            def _scale(g, _):
                vg = vals[b][pl.ds(g * L, L)]
                for i2 in range(L):
                    v = jnp.broadcast_to(vg[i2], (L,))
                    e = g * L + i2
                    for j in range(D // L):
                        rows[b][e, pl.ds(j * L, L)] = (
                            rows[b][e, pl.ds(j * L, L)] * v)
                return 0

            lax.fori_loop(0, CHUNK // L, _scale, 0)
            pltpu.sync_copy(rows[b], acc.at[ridx[b]], add=True)

            @pl.when(k + 2 < nch)
            def _():
                _issue_idx(k + 2, b)

        return 0

    lax.fori_loop(0, nch // 2, _pair, 0)
    plsc.subcore_barrier()

    @pl.when(on)
    def _():
        for t in range(-(-NDC // NS)):
            idx = s + t * NS

            @pl.when(idx < NDC)
            def _():
                r0 = idx * DRAIN
                pltpu.sync_copy(acc.at[pl.ds(r0, DRAIN)], obuf)
                pltpu.sync_copy(obuf, out.at[pl.ds(r0, DRAIN)])


def _spmm_sc(support, row3, col3, val3):
    mesh = plsc.VectorSubcoreMesh(core_axis_name="c", subcore_axis_name="s")
    fn = functools.partial(
        pl.kernel,
        out_type=jax.ShapeDtypeStruct((N, D), jnp.float32),
        mesh=mesh,
        scratch_types=(
            [pltpu.VMEM_SHARED((N, D), jnp.float32)]
            + [pltpu.VMEM((CHUNK,), jnp.int32)] * 2
            + [pltpu.VMEM((CHUNK,), jnp.int32)] * 2
            + [pltpu.VMEM((CHUNK,), jnp.float32)] * 2
            + [pltpu.VMEM((CHUNK, D), jnp.float32)] * 2
            + [pltpu.VMEM((DRAIN, D), jnp.float32)]
            + [pltpu.SemaphoreType.DMA] * 4
        ),
    )(_sc_body)
    return fn(support, row3, col3, val3)


def kernel(x, edge_index, edge_vals, W):
    support = _matmul(x, W)
    pad = EPAD - E
    row = jnp.pad(edge_index[0], (0, pad)).reshape(NS * NCHT, 1, CHUNK)
    col = jnp.pad(edge_index[1], (0, pad)).reshape(NS * NCHT, 1, CHUNK)
    val = jnp.pad(edge_vals, (0, pad)).reshape(NS * NCHT, 1, CHUNK)
    return _spmm_sc(support, row, col, val)


# speedup vs baseline: 1.2946x; 1.2946x over previous
import functools

import jax
import jax.numpy as jnp
from jax import lax
from jax.experimental import pallas as pl
from jax.experimental.pallas import tpu as pltpu
from jax.experimental.pallas import tpu_sc as plsc

N = 10000
E = 320000
D = 128
L = 16
NC = 2
NS = 16
NW = NC * NS
CHUNK = 128
NCH0 = 56
NCH1 = 102
NCHT = NCH0 + NCH1
EPAD = NS * NCHT * CHUNK
DRAIN = 80
NDC = N // DRAIN


def _mm_body(x_ref, w_ref, o_ref):
    o_ref[...] = jnp.dot(x_ref[...], w_ref[...],
                         preferred_element_type=jnp.float32)


def _matmul(x, W):
    return pl.pallas_call(
        _mm_body,
        grid=(10,),
        in_specs=[
            pl.BlockSpec((N // 10, D), lambda i: (i, 0)),
            pl.BlockSpec((D, D), lambda i: (0, 0)),
        ],
        out_specs=pl.BlockSpec((N // 10, D), lambda i: (i, 0)),
        out_shape=jax.ShapeDtypeStruct((N, D), jnp.float32),
    )(x, W)


def _add_body(a_ref, b_ref, o_ref):
    o_ref[...] = a_ref[...] + b_ref[...]


def _add(a, b):
    return pl.pallas_call(
        _add_body,
        grid=(10,),
        in_specs=[
            pl.BlockSpec((N // 10, D), lambda i: (i, 0)),
            pl.BlockSpec((N // 10, D), lambda i: (i, 0)),
        ],
        out_specs=pl.BlockSpec((N // 10, D), lambda i: (i, 0)),
        out_shape=jax.ShapeDtypeStruct((N, D), jnp.float32),
    )(a, b)


def _sc_body(support, row3, col3, val3, out, acc,
             cidx0, cidx1, ridx0, ridx1, vals0, vals1, rows0, rows1,
             obuf, semi0, semi1, semg0, semg1):
    c = lax.axis_index("c")
    s = lax.axis_index("s")
    nch = jnp.where(c == 0, NCH0, NCH1)
    base = c * NS * NCH0 + s * nch
    cidx = (cidx0, cidx1)
    ridx = (ridx0, ridx1)
    vals = (vals0, vals1)
    rows = (rows0, rows1)
    semi = (semi0, semi1)
    semg = (semg0, semg1)

    def _issue_idx(k, b):
        off = base + k
        pltpu.async_copy(col3.at[off, 0], cidx[b], semi[b])
        pltpu.async_copy(row3.at[off, 0], ridx[b], semi[b])
        pltpu.async_copy(val3.at[off, 0], vals[b], semi[b])

    def _wait_idx(b):
        pltpu.make_async_copy(col3.at[0, 0], cidx[b], semi[b]).wait()
        pltpu.make_async_copy(row3.at[0, 0], ridx[b], semi[b]).wait()
        pltpu.make_async_copy(val3.at[0, 0], vals[b], semi[b]).wait()

    def _issue_gather(b):
        pltpu.async_copy(support.at[cidx[b]], rows[b], semg[b])

    def _wait_gather(b):
        pltpu.make_async_copy(support.at[cidx[b]], rows[b], semg[b]).wait()

    _issue_idx(0, 0)
    _issue_idx(1, 1)
    _wait_idx(0)
    _issue_gather(0)

    zero = jnp.zeros((L,), jnp.float32)

    def _zrow(r, _):
        for j in range(D // L):
            obuf[r, pl.ds(j * L, L)] = zero
        return 0

    lax.fori_loop(0, DRAIN, _zrow, 0)
    for t in range(-(-NDC // NS)):
        idx = s + t * NS

        @pl.when(idx < NDC)
        def _():
            pltpu.sync_copy(obuf, acc.at[pl.ds(idx * DRAIN, DRAIN)])

    plsc.subcore_barrier()

    def _pair(i, _):
        for u in range(2):
            k = 2 * i + u
            b = u
            nb = 1 - u
            _wait_gather(b)

            @pl.when(k + 1 < nch)
            def _():
                _wait_idx(nb)
                _issue_gather(nb)

            def _scale(g, _):
                vg = vals[b][pl.ds(g * L, L)]
                for i2 in range(L):
                    v = jnp.broadcast_to(vg[i2], (L,))
                    e = g * L + i2
                    for j in range(D // L):
                        rows[b][e, pl.ds(j * L, L)] = (
                            rows[b][e, pl.ds(j * L, L)] * v)
                return 0

            lax.fori_loop(0, CHUNK // L, _scale, 0)
            pltpu.sync_copy(rows[b], acc.at[ridx[b]], add=True)

            @pl.when(k + 2 < nch)
            def _():
                _issue_idx(k + 2, b)

        return 0

    lax.fori_loop(0, nch // 2, _pair, 0)
    plsc.subcore_barrier()

    for t in range(-(-NDC // NS)):
        idx = s + t * NS

        @pl.when(idx < NDC)
        def _():
            r0 = idx * DRAIN
            pltpu.sync_copy(acc.at[pl.ds(r0, DRAIN)], obuf)
            pltpu.sync_copy(obuf, out.at[c, pl.ds(r0, DRAIN)])


def _spmm_sc(support, row3, col3, val3):
    mesh = plsc.VectorSubcoreMesh(core_axis_name="c", subcore_axis_name="s")
    fn = functools.partial(
        pl.kernel,
        out_type=jax.ShapeDtypeStruct((NC, N, D), jnp.float32),
        mesh=mesh,
        scratch_types=(
            [pltpu.VMEM_SHARED((N, D), jnp.float32)]
            + [pltpu.VMEM((CHUNK,), jnp.int32)] * 2
            + [pltpu.VMEM((CHUNK,), jnp.int32)] * 2
            + [pltpu.VMEM((CHUNK,), jnp.float32)] * 2
            + [pltpu.VMEM((CHUNK, D), jnp.float32)] * 2
            + [pltpu.VMEM((DRAIN, D), jnp.float32)]
            + [pltpu.SemaphoreType.DMA] * 4
        ),
    )(_sc_body)
    return fn(support, row3, col3, val3)


def kernel(x, edge_index, edge_vals, W):
    support = _matmul(x, W)
    pad = EPAD - E
    row = jnp.pad(edge_index[0], (0, pad)).reshape(NS * NCHT, 1, CHUNK)
    col = jnp.pad(edge_index[1], (0, pad)).reshape(NS * NCHT, 1, CHUNK)
    val = jnp.pad(edge_vals, (0, pad)).reshape(NS * NCHT, 1, CHUNK)
    partials = _spmm_sc(support, row, col, val)
    return _add(partials[0], partials[1])


# speedup vs baseline: 1.3425x; 1.0370x over previous
import functools

import jax
import jax.numpy as jnp
from jax import lax
from jax.experimental import pallas as pl
from jax.experimental.pallas import tpu as pltpu
from jax.experimental.pallas import tpu_sc as plsc

N = 10000
E = 320000
D = 128
L = 16
NC = 2
NS = 16
NW = NC * NS
CHUNK = 128
NCH0 = 66
NCH1 = 92
NCHT = NCH0 + NCH1
EPAD = NS * NCHT * CHUNK
DRAIN = 80
NDC = N // DRAIN


def _mm_body(x_ref, w_ref, o_ref):
    o_ref[...] = jnp.dot(x_ref[...], w_ref[...],
                         preferred_element_type=jnp.float32)


def _matmul(x, W):
    return pl.pallas_call(
        _mm_body,
        grid=(10,),
        in_specs=[
            pl.BlockSpec((N // 10, D), lambda i: (i, 0)),
            pl.BlockSpec((D, D), lambda i: (0, 0)),
        ],
        out_specs=pl.BlockSpec((N // 10, D), lambda i: (i, 0)),
        out_shape=jax.ShapeDtypeStruct((N, D), jnp.float32),
    )(x, W)


def _add_body(a_ref, b_ref, o_ref):
    o_ref[...] = a_ref[...] + b_ref[...]


def _add(a, b):
    return pl.pallas_call(
        _add_body,
        grid=(10,),
        in_specs=[
            pl.BlockSpec((N // 10, D), lambda i: (i, 0)),
            pl.BlockSpec((N // 10, D), lambda i: (i, 0)),
        ],
        out_specs=pl.BlockSpec((N // 10, D), lambda i: (i, 0)),
        out_shape=jax.ShapeDtypeStruct((N, D), jnp.float32),
    )(a, b)


def _sc_body(support, row3, col3, val3, out, acc,
             cidx0, cidx1, ridx0, ridx1, vals0, vals1, rows0, rows1,
             obuf, semi0, semi1, semg0, semg1):
    c = lax.axis_index("c")
    s = lax.axis_index("s")
    nch = jnp.where(c == 0, NCH0, NCH1)
    base = c * NS * NCH0 + s * nch
    cidx = (cidx0, cidx1)
    ridx = (ridx0, ridx1)
    vals = (vals0, vals1)
    rows = (rows0, rows1)
    semi = (semi0, semi1)
    semg = (semg0, semg1)

    def _issue_idx(k, b):
        off = base + k
        pltpu.async_copy(col3.at[off, 0], cidx[b], semi[b])
        pltpu.async_copy(row3.at[off, 0], ridx[b], semi[b])
        pltpu.async_copy(val3.at[off, 0], vals[b], semi[b])

    def _wait_idx(b):
        pltpu.make_async_copy(col3.at[0, 0], cidx[b], semi[b]).wait()
        pltpu.make_async_copy(row3.at[0, 0], ridx[b], semi[b]).wait()
        pltpu.make_async_copy(val3.at[0, 0], vals[b], semi[b]).wait()

    def _issue_gather(b):
        pltpu.async_copy(support.at[cidx[b]], rows[b], semg[b])

    def _wait_gather(b):
        pltpu.make_async_copy(support.at[cidx[b]], rows[b], semg[b]).wait()

    _issue_idx(0, 0)
    _issue_idx(1, 1)
    _wait_idx(0)
    _issue_gather(0)

    zero = jnp.zeros((L,), jnp.float32)

    def _zrow(r, _):
        for j in range(D // L):
            obuf[r, pl.ds(j * L, L)] = zero
        return 0

    lax.fori_loop(0, DRAIN, _zrow, 0)
    for t in range(-(-NDC // NS)):
        idx = s + t * NS

        @pl.when(idx < NDC)
        def _():
            pltpu.sync_copy(obuf, acc.at[pl.ds(idx * DRAIN, DRAIN)])

    plsc.subcore_barrier()

    def _pair(i, _):
        for u in range(2):
            k = 2 * i + u
            b = u
            nb = 1 - u
            _wait_gather(b)

            @pl.when(k + 1 < nch)
            def _():
                _wait_idx(nb)
                _issue_gather(nb)

            def _scale(g, _):
                vg = vals[b][pl.ds(g * L, L)]
                for i2 in range(L):
                    v = jnp.broadcast_to(vg[i2], (L,))
                    e = g * L + i2
                    for j in range(D // L):
                        rows[b][e, pl.ds(j * L, L)] = (
                            rows[b][e, pl.ds(j * L, L)] * v)
                return 0

            lax.fori_loop(0, CHUNK // L, _scale, 0)
            pltpu.sync_copy(rows[b], acc.at[ridx[b]], add=True)

            @pl.when(k + 2 < nch)
            def _():
                _issue_idx(k + 2, b)

        return 0

    lax.fori_loop(0, nch // 2, _pair, 0)
    plsc.subcore_barrier()

    for t in range(-(-NDC // NS)):
        idx = s + t * NS

        @pl.when(idx < NDC)
        def _():
            r0 = idx * DRAIN
            pltpu.sync_copy(acc.at[pl.ds(r0, DRAIN)], obuf)
            pltpu.sync_copy(obuf, out.at[c, pl.ds(r0, DRAIN)])


def _spmm_sc(support, row3, col3, val3):
    mesh = plsc.VectorSubcoreMesh(core_axis_name="c", subcore_axis_name="s")
    fn = functools.partial(
        pl.kernel,
        out_type=jax.ShapeDtypeStruct((NC, N, D), jnp.float32),
        mesh=mesh,
        scratch_types=(
            [pltpu.VMEM_SHARED((N, D), jnp.float32)]
            + [pltpu.VMEM((CHUNK,), jnp.int32)] * 2
            + [pltpu.VMEM((CHUNK,), jnp.int32)] * 2
            + [pltpu.VMEM((CHUNK,), jnp.float32)] * 2
            + [pltpu.VMEM((CHUNK, D), jnp.float32)] * 2
            + [pltpu.VMEM((DRAIN, D), jnp.float32)]
            + [pltpu.SemaphoreType.DMA] * 4
        ),
    )(_sc_body)
    return fn(support, row3, col3, val3)


def kernel(x, edge_index, edge_vals, W):
    support = _matmul(x, W)
    pad = EPAD - E
    row = jnp.pad(edge_index[0], (0, pad)).reshape(NS * NCHT, 1, CHUNK)
    col = jnp.pad(edge_index[1], (0, pad)).reshape(NS * NCHT, 1, CHUNK)
    val = jnp.pad(edge_vals, (0, pad)).reshape(NS * NCHT, 1, CHUNK)
    partials = _spmm_sc(support, row, col, val)
    return _add(partials[0], partials[1])
